# baseline (speedup 1.0000x reference)
import functools

import jax
import jax.numpy as jnp
from jax import lax
from jax.experimental import pallas as pl
from jax.experimental.pallas import tpu as pltpu
from jax.experimental.pallas import tpu_sc as plsc

B_SZ = 1024
OUT = 256
T = 17
PADC = 24
CEN = 120
NC, NS = 2, 16
NW = NC * NS
EPW = B_SZ // NW


def _floor_i32(v):
    ti = v.astype(jnp.int32)
    tf = ti.astype(jnp.float32)
    return jnp.where(v < tf, ti - 1, ti)


def _weights(f):
    f2 = f * f
    f3 = f2 * f
    g = 1.0 - f
    w0 = f3 * (1.0 / 6.0)
    w1 = -f3 * 0.5 + f2 * 0.5 + f * 0.5 + (1.0 / 6.0)
    w2 = f3 * 0.5 - f2 + (2.0 / 3.0)
    w3 = (g * g * g) * (1.0 / 6.0)
    return w0, w1, w2, w3


def _body(x_hbm, y_hbm, coeffs_hbm, out_hbm, x_v, y_v, coeffs_v, tmp_v,
          canvas_v, dma_sem):
    wid = lax.axis_index("s") * NC + lax.axis_index("c")
    base = wid * EPW

    pltpu.sync_copy(x_hbm.at[pl.ds(base, EPW)], x_v)
    pltpu.sync_copy(y_hbm.at[pl.ds(base, EPW)], y_v)
    pltpu.sync_copy(coeffs_hbm, coeffs_v)

    zvec = jnp.zeros((16,), jnp.float32)
    lane = lax.iota(jnp.int32, 16)

    def zero_canvas_row(r, c):
        for i in range(OUT // 16):
            canvas_v[r, pl.ds(i * 16, 16)] = zvec
        return c

    lax.fori_loop(0, OUT, zero_canvas_row, 0)

    def offsets(e):
        idx = jnp.full((16,), e, jnp.int32)
        xv = plsc.load_gather(x_v, [idx])
        yv = plsc.load_gather(y_v, [idx])
        sx_i = _floor_i32(xv)
        sy_i = _floor_i32(yv)
        return xv, yv, sx_i, sy_i

    def rowblend(e):
        xv, yv, sx_i, sy_i = offsets(e)
        bx = _weights(xv - sx_i.astype(jnp.float32))
        by = _weights(yv - sy_i.astype(jnp.float32))
        for r in range(T):
            for off in (0, 8):
                acc = bx[0] * coeffs_v[r + 2, pl.ds(off, 16)]
                for a in (1, 2, 3):
                    acc = acc + bx[a] * coeffs_v[r + 2 + a, pl.ds(off, 16)]
                tmp_v[r, pl.ds(off, 16)] = acc
        return by, sx_i + CEN, sy_i + CEN

    def col_indices(c0):
        cidx = [None, None]
        cmask = [None, None]
        for off in (0, 1):
            cc = c0 + off + lane
            inb = (cc >= 0) & (cc < OUT)
            if off == 1:
                inb = inb & (lane < T - 1)
            cidx[off] = jnp.clip(cc, 0, OUT - 1)
            cmask[off] = inb
        return cidx, cmask

    def colblend_scatter(by, r0, c0):
        cidx, cmask = col_indices(c0)
        for r in range(T):
            rr = r0 + r
            rin = (rr >= 0) & (rr < OUT)
            ridx = jnp.clip(rr, 0, OUT - 1)
            for off in (0, 1):
                acc = by[0] * tmp_v[r, pl.ds(2 + off, 16)]
                for b in (1, 2, 3):
                    acc = acc + by[b] * tmp_v[r, pl.ds(2 + b + off, 16)]
                plsc.store_scatter(canvas_v, [ridx, cidx[off]], acc,
                                   mask=cmask[off] & rin)

    def rezero(e):
        _, _, sx_i, sy_i = offsets(e)
        r0 = sx_i + CEN
        cidx, _ = col_indices(sy_i + CEN)
        for r in range(T):
            ridx = jnp.clip(r0 + r, 0, OUT - 1)
            for off in (0, 1):
                plsc.store_scatter(canvas_v, [ridx, cidx[off]], zvec)

    def element(e, c):
        by, r0, c0 = rowblend(e)
        colblend_scatter(by, r0, c0)
        cp = pltpu.async_copy(canvas_v, out_hbm.at[base + e, 0], dma_sem)
        cp.wait()
        rezero(e)
        return c

    lax.fori_loop(0, EPW, element, 0)


@jax.jit
def _spline2d(x1, y1, coeffs):
    mesh = plsc.VectorSubcoreMesh(core_axis_name="c", subcore_axis_name="s")
    run = functools.partial(
        pl.kernel,
        out_type=jax.ShapeDtypeStruct((B_SZ, 1, OUT, OUT), jnp.float32),
        mesh=mesh,
        compiler_params=pltpu.CompilerParams(use_tc_tiling_on_sc=False,
                                             needs_layout_passes=False),
        scratch_types=[
            pltpu.VMEM((EPW,), jnp.float32),
            pltpu.VMEM((EPW,), jnp.float32),
            pltpu.VMEM((PADC, PADC), jnp.float32),
            pltpu.VMEM((T, PADC), jnp.float32),
            pltpu.VMEM((OUT, OUT), jnp.float32),
            pltpu.SemaphoreType.DMA,
        ],
    )(_body)
    return run(x1, y1, coeffs)


def kernel(x, y, coeffs):
    return _spline2d(x.reshape(B_SZ), y.reshape(B_SZ), coeffs)


# speedup vs baseline: 3.3934x; 3.3934x over previous
import functools

import jax
import jax.numpy as jnp
from jax import lax
from jax.experimental import pallas as pl
from jax.experimental.pallas import tpu as pltpu
from jax.experimental.pallas import tpu_sc as plsc

B_SZ = 1024
OUT = 256
T = 17
PADC = 24
CEN = 120
NC, NS = 2, 16
NW = NC * NS
EPW = B_SZ // NW


def _floor_i32(v):
    ti = v.astype(jnp.int32)
    tf = ti.astype(jnp.float32)
    return jnp.where(v < tf, ti - 1, ti)


def _weights(f):
    f2 = f * f
    f3 = f2 * f
    g = 1.0 - f
    w0 = f3 * (1.0 / 6.0)
    w1 = -f3 * 0.5 + f2 * 0.5 + f * 0.5 + (1.0 / 6.0)
    w2 = f3 * 0.5 - f2 + (2.0 / 3.0)
    w3 = (g * g * g) * (1.0 / 6.0)
    return w0, w1, w2, w3


def _body(x_hbm, y_hbm, coeffs_hbm, out_hbm, x_v, y_v, coeffs_v, tmp_v,
          canvas_v, dma_sem):
    wid = lax.axis_index("s") * NC + lax.axis_index("c")
    base = wid * EPW

    pltpu.sync_copy(x_hbm.at[pl.ds(base, EPW)], x_v)
    pltpu.sync_copy(y_hbm.at[pl.ds(base, EPW)], y_v)
    pltpu.sync_copy(coeffs_hbm, coeffs_v)

    zvec = jnp.zeros((16,), jnp.float32)
    lane = lax.iota(jnp.int32, 16)

    def zero_canvas_row(r, c):
        for i in range(OUT // 16):
            canvas_v[r, pl.ds(i * 16, 16)] = zvec
        return c

    lax.fori_loop(0, OUT, zero_canvas_row, 0)

    def offsets(e):
        idx = jnp.full((16,), e, jnp.int32)
        xv = plsc.load_gather(x_v, [idx])
        yv = plsc.load_gather(y_v, [idx])
        sx_i = _floor_i32(xv)
        sy_i = _floor_i32(yv)
        return xv, yv, sx_i, sy_i

    def rowblend(e):
        xv, yv, sx_i, sy_i = offsets(e)
        bx = _weights(xv - sx_i.astype(jnp.float32))
        by = _weights(yv - sy_i.astype(jnp.float32))
        for r in range(T):
            for off in (0, 8):
                acc = bx[0] * coeffs_v[r + 2, pl.ds(off, 16)]
                for a in (1, 2, 3):
                    acc = acc + bx[a] * coeffs_v[r + 2 + a, pl.ds(off, 16)]
                tmp_v[r, pl.ds(off, 16)] = acc
        return by, sx_i + CEN, sy_i + CEN

    def col_indices(c0):
        cidx = [None, None]
        cmask = [None, None]
        for off in (0, 1):
            cc = c0 + off + lane
            inb = (cc >= 0) & (cc < OUT)
            if off == 1:
                inb = inb & (lane < T - 1)
            cidx[off] = jnp.clip(cc, 0, OUT - 1)
            cmask[off] = inb
        return cidx, cmask

    def colblend_scatter(by, r0, c0):
        cidx, cmask = col_indices(c0)
        for r in range(T):
            rr = r0 + r
            rin = (rr >= 0) & (rr < OUT)
            ridx = jnp.clip(rr, 0, OUT - 1)
            for off in (0, 1):
                acc = by[0] * tmp_v[r, pl.ds(2 + off, 16)]
                for b in (1, 2, 3):
                    acc = acc + by[b] * tmp_v[r, pl.ds(2 + b + off, 16)]
                plsc.store_scatter(canvas_v, [ridx, cidx[off]], acc,
                                   mask=cmask[off] & rin)

    def rezero(e):
        _, _, sx_i, sy_i = offsets(e)
        r0 = sx_i + CEN
        cidx, _ = col_indices(sy_i + CEN)
        for r in range(T):
            ridx = jnp.clip(r0 + r, 0, OUT - 1)
            for off in (0, 1):
                plsc.store_scatter(canvas_v, [ridx, cidx[off]], zvec)

    def element(e, c):
        by, r0, c0 = rowblend(e)
        colblend_scatter(by, r0, c0)
        cp = pltpu.async_copy(canvas_v, out_hbm.at[base + e, 0], dma_sem)
        cp.wait()
        rezero(e)
        return c

    lax.fori_loop(0, EPW, element, 0)


@jax.jit
def _spline2d(x1, y1, coeffs):
    mesh = plsc.VectorSubcoreMesh(core_axis_name="c", subcore_axis_name="s")
    run = functools.partial(
        pl.kernel,
        out_type=jax.ShapeDtypeStruct((B_SZ, 1, OUT, OUT), jnp.float32),
        mesh=mesh,
        compiler_params=pltpu.CompilerParams(use_tc_tiling_on_sc=True,
                                             needs_layout_passes=False),
        scratch_types=[
            pltpu.VMEM((EPW,), jnp.float32),
            pltpu.VMEM((EPW,), jnp.float32),
            pltpu.VMEM((PADC, PADC), jnp.float32),
            pltpu.VMEM((T, PADC), jnp.float32),
            pltpu.VMEM((OUT, OUT), jnp.float32),
            pltpu.SemaphoreType.DMA,
        ],
    )(_body)
    return run(x1, y1, coeffs)


def kernel(x, y, coeffs):
    return _spline2d(x.reshape(B_SZ), y.reshape(B_SZ), coeffs)


# speedup vs baseline: 3.3975x; 1.0012x over previous
import functools

import jax
import jax.numpy as jnp
from jax import lax
from jax.experimental import pallas as pl
from jax.experimental.pallas import tpu as pltpu
from jax.experimental.pallas import tpu_sc as plsc

B_SZ = 1024
OUT = 256
T = 17
PADC = 24
CEN = 120
NC, NS = 2, 16
NW = NC * NS
EPW = B_SZ // NW


def _floor_i32(v):
    ti = v.astype(jnp.int32)
    tf = ti.astype(jnp.float32)
    return jnp.where(v < tf, ti - 1, ti)


def _weights(f):
    f2 = f * f
    f3 = f2 * f
    g = 1.0 - f
    w0 = f3 * (1.0 / 6.0)
    w1 = -f3 * 0.5 + f2 * 0.5 + f * 0.5 + (1.0 / 6.0)
    w2 = f3 * 0.5 - f2 + (2.0 / 3.0)
    w3 = (g * g * g) * (1.0 / 6.0)
    return w0, w1, w2, w3


def _body(x_hbm, y_hbm, coeffs_hbm, out_hbm, x_v, y_v, coeffs_v, tmp_v,
          canvas_v, dma_sem, sem2):
    wid = lax.axis_index("s") * NC + lax.axis_index("c")
    base = wid * EPW

    pltpu.sync_copy(x_hbm.at[pl.ds(base, EPW)], x_v)
    pltpu.sync_copy(y_hbm.at[pl.ds(base, EPW)], y_v)
    pltpu.sync_copy(coeffs_hbm, coeffs_v)

    zvec = jnp.zeros((16,), jnp.float32)
    lane = lax.iota(jnp.int32, 16)

    def zero_canvas_row(r, c):
        for i in range(OUT // 16):
            canvas_v[r, pl.ds(i * 16, 16)] = zvec
        return c

    lax.fori_loop(0, OUT, zero_canvas_row, 0)

    def offsets(e):
        idx = jnp.full((16,), e, jnp.int32)
        xv = plsc.load_gather(x_v, [idx])
        yv = plsc.load_gather(y_v, [idx])
        sx_i = _floor_i32(xv)
        sy_i = _floor_i32(yv)
        return xv, yv, sx_i, sy_i

    def rowblend(e):
        xv, yv, sx_i, sy_i = offsets(e)
        bx = _weights(xv - sx_i.astype(jnp.float32))
        by = _weights(yv - sy_i.astype(jnp.float32))
        for r in range(T):
            for off in (0, 8):
                acc = bx[0] * coeffs_v[r + 2, pl.ds(off, 16)]
                for a in (1, 2, 3):
                    acc = acc + bx[a] * coeffs_v[r + 2 + a, pl.ds(off, 16)]
                tmp_v[r, pl.ds(off, 16)] = acc
        return by, sx_i + CEN, sy_i + CEN

    def col_indices(c0):
        cidx = [None, None]
        cmask = [None, None]
        for off in (0, 1):
            cc = c0 + off + lane
            inb = (cc >= 0) & (cc < OUT)
            if off == 1:
                inb = inb & (lane < T - 1)
            cidx[off] = jnp.clip(cc, 0, OUT - 1)
            cmask[off] = inb
        return cidx, cmask

    def colblend_scatter(by, r0, c0):
        cidx, cmask = col_indices(c0)
        for r in range(T):
            rr = r0 + r
            rin = (rr >= 0) & (rr < OUT)
            ridx = jnp.clip(rr, 0, OUT - 1)
            for off in (0, 1):
                acc = by[0] * tmp_v[r, pl.ds(2 + off, 16)]
                for b in (1, 2, 3):
                    acc = acc + by[b] * tmp_v[r, pl.ds(2 + b + off, 16)]
                plsc.store_scatter(canvas_v, [ridx, cidx[off]], acc,
                                   mask=cmask[off] & rin)

    def rezero(e):
        _, _, sx_i, sy_i = offsets(e)
        r0 = sx_i + CEN
        cidx, _ = col_indices(sy_i + CEN)
        for r in range(T):
            ridx = jnp.clip(r0 + r, 0, OUT - 1)
            for off in (0, 1):
                plsc.store_scatter(canvas_v, [ridx, cidx[off]], zvec)

    H = OUT // 2

    def element(e, c):
        by, r0, c0 = rowblend(e)
        colblend_scatter(by, r0, c0)
        cp1 = pltpu.async_copy(canvas_v.at[pl.ds(0, H)],
                               out_hbm.at[base + e, 0, pl.ds(0, H)], dma_sem)
        cp2 = pltpu.async_copy(canvas_v.at[pl.ds(H, H)],
                               out_hbm.at[base + e, 0, pl.ds(H, H)], sem2)
        cp1.wait()
        cp2.wait()
        rezero(e)
        return c

    lax.fori_loop(0, EPW, element, 0)


@jax.jit
def _spline2d(x1, y1, coeffs):
    mesh = plsc.VectorSubcoreMesh(core_axis_name="c", subcore_axis_name="s")
    run = functools.partial(
        pl.kernel,
        out_type=jax.ShapeDtypeStruct((B_SZ, 1, OUT, OUT), jnp.float32),
        mesh=mesh,
        compiler_params=pltpu.CompilerParams(use_tc_tiling_on_sc=True,
                                             needs_layout_passes=False),
        scratch_types=[
            pltpu.VMEM((EPW,), jnp.float32),
            pltpu.VMEM((EPW,), jnp.float32),
            pltpu.VMEM((PADC, PADC), jnp.float32),
            pltpu.VMEM((T, PADC), jnp.float32),
            pltpu.VMEM((OUT, OUT), jnp.float32),
            pltpu.SemaphoreType.DMA,
            pltpu.SemaphoreType.DMA,
        ],
    )(_body)
    return run(x1, y1, coeffs)


def kernel(x, y, coeffs):
    return _spline2d(x.reshape(B_SZ), y.reshape(B_SZ), coeffs)


# speedup vs baseline: 3.4296x; 1.0094x over previous
import functools

import jax
import jax.numpy as jnp
from jax import lax
from jax.experimental import pallas as pl
from jax.experimental.pallas import tpu as pltpu
from jax.experimental.pallas import tpu_sc as plsc

B_SZ = 1024
OUT = 256
T = 17
PADC = 24
CEN = 120
NC, NS = 2, 16
NW = NC * NS
EPW = B_SZ // NW


def _floor_i32(v):
    ti = v.astype(jnp.int32)
    tf = ti.astype(jnp.float32)
    return jnp.where(v < tf, ti - 1, ti)


def _weights(f):
    f2 = f * f
    f3 = f2 * f
    g = 1.0 - f
    w0 = f3 * (1.0 / 6.0)
    w1 = -f3 * 0.5 + f2 * 0.5 + f * 0.5 + (1.0 / 6.0)
    w2 = f3 * 0.5 - f2 + (2.0 / 3.0)
    w3 = (g * g * g) * (1.0 / 6.0)
    return w0, w1, w2, w3


def _body(x_hbm, y_hbm, coeffs_hbm, out_hbm, x_v, y_v, coeffs_v, tmp_v,
          canvas_v, dma_sem, sem2):
    wid = lax.axis_index("s") * NC + lax.axis_index("c")
    base = wid * EPW

    pltpu.sync_copy(x_hbm.at[pl.ds(base, EPW)], x_v)
    pltpu.sync_copy(y_hbm.at[pl.ds(base, EPW)], y_v)
    pltpu.sync_copy(coeffs_hbm, coeffs_v)

    zvec = jnp.zeros((16,), jnp.float32)
    lane = lax.iota(jnp.int32, 16)

    def zero_canvas_row(r, c):
        for i in range(OUT // 16):
            canvas_v[r, pl.ds(i * 16, 16)] = zvec
        return c

    lax.fori_loop(0, OUT, zero_canvas_row, 0)

    def offsets(e):
        idx = jnp.full((16,), e, jnp.int32)
        xv = plsc.load_gather(x_v, [idx])
        yv = plsc.load_gather(y_v, [idx])
        sx_i = _floor_i32(xv)
        sy_i = _floor_i32(yv)
        return xv, yv, sx_i, sy_i

    def rowblend(e):
        xv, yv, sx_i, sy_i = offsets(e)
        bx = _weights(xv - sx_i.astype(jnp.float32))
        by = _weights(yv - sy_i.astype(jnp.float32))
        for r in range(T):
            for off in (0, 8):
                acc = bx[0] * coeffs_v[r + 2, pl.ds(off, 16)]
                for a in (1, 2, 3):
                    acc = acc + bx[a] * coeffs_v[r + 2 + a, pl.ds(off, 16)]
                tmp_v[r, pl.ds(off, 16)] = acc
        return by, sx_i + CEN, sy_i + CEN

    def col_indices(c0):
        cidx = [None, None]
        cmask = [None, None]
        for off in (0, 1):
            cc = c0 + off + lane
            inb = (cc >= 0) & (cc < OUT)
            if off == 1:
                inb = inb & (lane < T - 1)
            cidx[off] = jnp.clip(cc, 0, OUT - 1)
            cmask[off] = inb
        return cidx, cmask

    def colblend_scatter(by, r0, c0):
        cidx, cmask = col_indices(c0)
        for r in range(T):
            rr = r0 + r
            rin = (rr >= 0) & (rr < OUT)
            ridx = jnp.clip(rr, 0, OUT - 1)
            for off in (0, 1):
                acc = by[0] * tmp_v[r, pl.ds(2 + off, 16)]
                for b in (1, 2, 3):
                    acc = acc + by[b] * tmp_v[r, pl.ds(2 + b + off, 16)]
                plsc.store_scatter(canvas_v, [ridx, cidx[off]], acc,
                                   mask=cmask[off] & rin)

    def rezero(e):
        _, _, sx_i, sy_i = offsets(e)
        r0 = sx_i + CEN
        cidx, _ = col_indices(sy_i + CEN)
        for r in range(T):
            ridx = jnp.clip(r0 + r, 0, OUT - 1)
            for off in (0, 1):
                plsc.store_scatter(canvas_v, [ridx, cidx[off]], zvec)

    H = OUT // 2

    def element(e, carry):
        by = carry[:4]
        r0, c0 = carry[4], carry[5]
        colblend_scatter(by, r0, c0)
        cp1 = pltpu.async_copy(canvas_v.at[pl.ds(0, H)],
                               out_hbm.at[base + e, 0, pl.ds(0, H)], dma_sem)
        cp2 = pltpu.async_copy(canvas_v.at[pl.ds(H, H)],
                               out_hbm.at[base + e, 0, pl.ds(H, H)], sem2)
        byn, r0n, c0n = rowblend(jnp.minimum(e + 1, EPW - 1))
        cp1.wait()
        cp2.wait()
        rezero(e)
        return (byn[0], byn[1], byn[2], byn[3], r0n, c0n)

    by0, r00, c00 = rowblend(0)
    lax.fori_loop(0, EPW, element,
                  (by0[0], by0[1], by0[2], by0[3], r00, c00))


@jax.jit
def _spline2d(x1, y1, coeffs):
    mesh = plsc.VectorSubcoreMesh(core_axis_name="c", subcore_axis_name="s")
    run = functools.partial(
        pl.kernel,
        out_type=jax.ShapeDtypeStruct((B_SZ, 1, OUT, OUT), jnp.float32),
        mesh=mesh,
        compiler_params=pltpu.CompilerParams(use_tc_tiling_on_sc=True,
                                             needs_layout_passes=False),
        scratch_types=[
            pltpu.VMEM((EPW,), jnp.float32),
            pltpu.VMEM((EPW,), jnp.float32),
            pltpu.VMEM((PADC, PADC), jnp.float32),
            pltpu.VMEM((T, PADC), jnp.float32),
            pltpu.VMEM((OUT, OUT), jnp.float32),
            pltpu.SemaphoreType.DMA,
            pltpu.SemaphoreType.DMA,
        ],
    )(_body)
    return run(x1, y1, coeffs)


def kernel(x, y, coeffs):
    return _spline2d(x.reshape(B_SZ), y.reshape(B_SZ), coeffs)
